# baseline (speedup 1.0000x reference)
import functools

import jax
import jax.numpy as jnp
from jax import lax
from jax.experimental import pallas as pl
from jax.experimental.pallas import tpu as pltpu
from jax.experimental.pallas import tpu_sc as plsc

_BATCH = 4096
_SEQ = 200
_DIM = 32
_VOCAB = 1000000
_TPR = 4
_NGRP = _VOCAB // _TPR

_NC = 2
_NS = 16
_NW = _NC * _NS
_ROWS_W = _BATCH // _NW
_BLK = 8
_NBLK = _ROWS_W // _BLK

_L = 16
_NVEC = 13


@functools.partial(
    pl.kernel,
    mesh=plsc.VectorSubcoreMesh(core_axis_name="c", subcore_axis_name="s"),
    out_type=jax.ShapeDtypeStruct((_BATCH, _SEQ, _DIM), jnp.float32),
    scratch_types=[
        pltpu.VMEM((_BLK, _SEQ), jnp.int32),
        pltpu.VMEM((_SEQ,), jnp.int32),
        pltpu.VMEM((_SEQ,), jnp.int32),
        pltpu.VMEM((_SEQ, _TPR * _DIM), jnp.float32),
        pltpu.VMEM((_SEQ, _DIM), jnp.float32),
        pltpu.SemaphoreType.DMA,
    ],
    compiler_params=pltpu.CompilerParams(needs_layout_passes=False),
)
def _gather_kernel(x_hbm, table_hbm, out_hbm,
                   xblk_v, grp_v, off_v, tiles_v, rows_v, gsem):
    wid = lax.axis_index("s") * _NC + lax.axis_index("c")
    base = wid * _ROWS_W

    def blk_body(k, carry):
        b0 = base + k * _BLK
        pltpu.sync_copy(x_hbm.at[pl.ds(b0, _BLK)], xblk_v)

        def row_body(r, c1):
            def split(g, c2):
                o = jnp.minimum(g * _L, _SEQ - _L)
                v = xblk_v[r, pl.ds(o, _L)]
                grp_v[pl.ds(o, _L)] = lax.shift_right_logical(v, 2)
                off_v[pl.ds(o, _L)] = lax.bitwise_and(v, 3) * _DIM
                return c2

            lax.fori_loop(0, _NVEC, split, 0)
            pltpu.async_copy(table_hbm.at[grp_v], tiles_v, gsem).wait()

            def select(g, c2):
                o = jnp.minimum(g * _L, _SEQ - _L)
                iv = o + lax.iota(jnp.int32, 16)
                jv = off_v[pl.ds(o, _L)]
                for c in range(_DIM):
                    cv = jnp.full((_L,), c, jnp.int32)
                    val = plsc.load_gather(tiles_v, [iv, jv + c])
                    plsc.store_scatter(rows_v, [iv, cv], val)
                return c2

            lax.fori_loop(0, _NVEC, select, 0)
            pltpu.sync_copy(rows_v, out_hbm.at[b0 + r])
            return c1

        lax.fori_loop(0, _BLK, row_body, 0)
        return carry

    lax.fori_loop(0, _NBLK, blk_body, 0)


def kernel(X, W):
    W_packed = W.reshape(_NGRP, _TPR * _DIM)
    return _gather_kernel(X.astype(jnp.int32), W_packed)


# speedup vs baseline: 1.7407x; 1.7407x over previous
import functools

import jax
import jax.numpy as jnp
from jax import lax
from jax.experimental import pallas as pl
from jax.experimental.pallas import tpu as pltpu
from jax.experimental.pallas import tpu_sc as plsc

_BATCH = 4096
_SEQ = 200
_DIM = 32

_NC = 2
_NS = 16
_NW = _NC * _NS
_BB = _BATCH // _NW

_L = 16
_JB = _BB // _L


@functools.partial(
    pl.kernel,
    mesh=plsc.VectorSubcoreMesh(core_axis_name="c", subcore_axis_name="s"),
    out_type=jax.ShapeDtypeStruct((_SEQ * _DIM, _BATCH), jnp.float32),
    scratch_types=[
        pltpu.VMEM((_SEQ, _BB), jnp.int32),
        pltpu.VMEM((2, _BB, _DIM), jnp.float32),
        pltpu.VMEM((2, _DIM, _BB), jnp.float32),
        pltpu.SemaphoreType.DMA,
        pltpu.SemaphoreType.DMA,
        pltpu.SemaphoreType.DMA,
    ],
    compiler_params=pltpu.CompilerParams(
        use_tc_tiling_on_sc=False, needs_layout_passes=False),
)
def _gather_kernel(idx_hbm, table_hbm, out_hbm,
                   idx_v, rows_v, tr_v, gsem, osem, isem):
    wid = lax.axis_index("s") * _NC + lax.axis_index("c")
    b0 = wid * _BB

    pltpu.async_copy(idx_hbm.at[:, pl.ds(b0, _BB)], idx_v, isem).wait()

    pltpu.async_copy(table_hbm.at[idx_v.at[0]], rows_v.at[0], gsem)

    def step(t, carry):
        buf = lax.rem(t, 2)
        nbuf = 1 - buf
        pltpu.make_async_copy(table_hbm.at[idx_v.at[t]],
                              rows_v.at[buf], gsem).wait()

        @pl.when(t + 1 < _SEQ)
        def _():
            pltpu.async_copy(table_hbm.at[idx_v.at[t + 1]],
                             rows_v.at[nbuf], gsem)

        def tr_d(d, c2):
            for g in range(_JB):
                jv = g * _L + lax.iota(jnp.int32, 16)
                dv = jnp.full((_L,), 0, jnp.int32) + d
                val = plsc.load_gather(rows_v.at[buf], [jv, dv])
                tr_v[buf, d, pl.ds(g * _L, _L)] = val
            return c2

        lax.fori_loop(0, _DIM, tr_d, 0)

        @pl.when(t >= 2)
        def _():
            pltpu.make_async_copy(
                tr_v.at[buf],
                out_hbm.at[pl.ds((t - 2) * _DIM, _DIM), pl.ds(b0, _BB)],
                osem).wait()

        pltpu.async_copy(
            tr_v.at[buf],
            out_hbm.at[pl.ds(t * _DIM, _DIM), pl.ds(b0, _BB)], osem)
        return carry

    lax.fori_loop(0, _SEQ, step, 0)

    pltpu.make_async_copy(
        tr_v.at[0],
        out_hbm.at[pl.ds((_SEQ - 2) * _DIM, _DIM), pl.ds(b0, _BB)],
        osem).wait()
    pltpu.make_async_copy(
        tr_v.at[1],
        out_hbm.at[pl.ds((_SEQ - 1) * _DIM, _DIM), pl.ds(b0, _BB)],
        osem).wait()


def kernel(X, W):
    idx_t = X.astype(jnp.int32).T
    out2 = _gather_kernel(idx_t, W)
    return out2.reshape(_SEQ, _DIM, _BATCH).transpose(2, 0, 1)


# speedup vs baseline: 1.8788x; 1.0794x over previous
import functools

import jax
import jax.numpy as jnp
from jax import lax
from jax.experimental import pallas as pl
from jax.experimental.pallas import tpu as pltpu
from jax.experimental.pallas import tpu_sc as plsc

_BATCH = 4096
_SEQ = 200
_DIM = 32

_NC = 2
_NS = 16
_NW = _NC * _NS
_BB = _BATCH // _NW

_L = 16
_JB = _BB // _L


@functools.partial(
    pl.kernel,
    mesh=plsc.VectorSubcoreMesh(core_axis_name="c", subcore_axis_name="s"),
    out_type=jax.ShapeDtypeStruct((_SEQ, _DIM // 8, _BATCH // _BB, 8, _BB),
                                  jnp.float32),
    scratch_types=[
        pltpu.VMEM((_SEQ, _BB), jnp.int32),
        pltpu.VMEM((_SEQ, _BB), jnp.int32),
        pltpu.VMEM((2, _BB, 128), jnp.float32),
        pltpu.VMEM((2, _DIM // 8, 8, _BB), jnp.float32),
        pltpu.SemaphoreType.DMA,
        pltpu.SemaphoreType.DMA,
        pltpu.SemaphoreType.DMA,
    ],
    compiler_params=pltpu.CompilerParams(
        use_tc_tiling_on_sc=False, needs_layout_passes=False),
)
def _gather_kernel(idxs_hbm, idxm_hbm, table_hbm, out_hbm,
                   idxs_v, idxm_v, rows_v, tr_v, gsem, osem, isem):
    wid = lax.axis_index("s") * _NC + lax.axis_index("c")
    b0 = wid * _BB

    pltpu.async_copy(idxs_hbm.at[:, pl.ds(b0, _BB)], idxs_v, isem)
    pltpu.async_copy(idxm_hbm.at[:, pl.ds(b0, _BB)], idxm_v, isem)
    pltpu.make_async_copy(idxs_hbm.at[:, pl.ds(b0, _BB)], idxs_v, isem).wait()
    pltpu.make_async_copy(idxm_hbm.at[:, pl.ds(b0, _BB)], idxm_v, isem).wait()

    pltpu.async_copy(table_hbm.at[idxs_v.at[0]], rows_v.at[0], gsem)

    def step(t, carry):
        buf = lax.rem(t, 2)
        nbuf = 1 - buf
        pltpu.make_async_copy(table_hbm.at[idxs_v.at[t]],
                              rows_v.at[buf], gsem).wait()

        @pl.when(t + 1 < _SEQ)
        def _():
            pltpu.async_copy(table_hbm.at[idxs_v.at[t + 1]],
                             rows_v.at[nbuf], gsem)

        for g in range(_JB):
            jv = g * _L + lax.iota(jnp.int32, _L)
            off = idxm_v[t, pl.ds(g * _L, _L)] * _DIM

            for d in range(_DIM):
                val = plsc.load_gather(rows_v.at[buf], [jv, off + d])
                tr_v[buf, d // 8, d % 8, pl.ds(g * _L, _L)] = val

        @pl.when(t >= 2)
        def _():
            pltpu.make_async_copy(
                tr_v.at[buf], out_hbm.at[t - 2, :, wid], osem).wait()

        pltpu.async_copy(tr_v.at[buf], out_hbm.at[t, :, wid], osem)
        return carry

    lax.fori_loop(0, _SEQ, step, 0)

    pltpu.make_async_copy(
        tr_v.at[0], out_hbm.at[_SEQ - 2, :, wid], osem).wait()
    pltpu.make_async_copy(
        tr_v.at[1], out_hbm.at[_SEQ - 1, :, wid], osem).wait()


def kernel(X, W):
    xi = X.astype(jnp.int32)
    idxs = (xi >> 2).T
    idxm = (xi & 3).T
    w4 = W.reshape(_BATCH // _BATCH * 250000, 128)
    out5 = _gather_kernel(idxs, idxm, w4)
    return out5.transpose(2, 4, 0, 1, 3).reshape(_BATCH, _SEQ, _DIM)


# speedup vs baseline: 1.8810x; 1.0011x over previous
import functools

import jax
import jax.numpy as jnp
from jax import lax
from jax.experimental import pallas as pl
from jax.experimental.pallas import tpu as pltpu
from jax.experimental.pallas import tpu_sc as plsc

_BATCH = 4096
_SEQ = 200
_DIM = 32

_NC = 2
_NS = 16
_NW = _NC * _NS
_BB = _BATCH // _NW

_L = 16
_JB = _BB // _L


@functools.partial(
    pl.kernel,
    mesh=plsc.VectorSubcoreMesh(core_axis_name="c", subcore_axis_name="s"),
    out_type=jax.ShapeDtypeStruct((_SEQ, _DIM // 8, _BATCH // _BB, 8, _BB),
                                  jnp.float32),
    scratch_types=[
        pltpu.VMEM((_SEQ, _BB), jnp.int32),
        pltpu.VMEM((_SEQ, _BB), jnp.int32),
        pltpu.VMEM((4, _BB, 128), jnp.float32),
        pltpu.VMEM((2, _DIM // 8, 8, _BB), jnp.float32),
        pltpu.SemaphoreType.DMA,
        pltpu.SemaphoreType.DMA,
        pltpu.SemaphoreType.DMA,
    ],
    compiler_params=pltpu.CompilerParams(
        use_tc_tiling_on_sc=False, needs_layout_passes=False),
)
def _gather_kernel(idxs_hbm, idxm_hbm, table_hbm, out_hbm,
                   idxs_v, idxm_v, rows_v, tr_v, gsem, osem, isem):
    wid = lax.axis_index("s") * _NC + lax.axis_index("c")
    b0 = wid * _BB

    pltpu.async_copy(idxs_hbm.at[:, pl.ds(b0, _BB)], idxs_v, isem)
    pltpu.async_copy(idxm_hbm.at[:, pl.ds(b0, _BB)], idxm_v, isem)
    pltpu.make_async_copy(idxs_hbm.at[:, pl.ds(b0, _BB)], idxs_v, isem).wait()
    pltpu.make_async_copy(idxm_hbm.at[:, pl.ds(b0, _BB)], idxm_v, isem).wait()

    for tt in range(3):
        pltpu.async_copy(table_hbm.at[idxs_v.at[tt]], rows_v.at[tt], gsem)

    def step(t, carry):
        buf = lax.rem(t, 4)
        obuf = lax.rem(t, 2)
        pltpu.make_async_copy(table_hbm.at[idxs_v.at[t]],
                              rows_v.at[buf], gsem).wait()

        @pl.when(t + 3 < _SEQ)
        def _():
            pltpu.async_copy(table_hbm.at[idxs_v.at[t + 3]],
                             rows_v.at[lax.rem(t + 3, 4)], gsem)

        @pl.when(t >= 2)
        def _():
            pltpu.make_async_copy(
                tr_v.at[obuf], out_hbm.at[t - 2, :, wid], osem).wait()

        for g in range(_JB):
            jv = g * _L + lax.iota(jnp.int32, _L)
            off = idxm_v[t, pl.ds(g * _L, _L)] * _DIM

            for d in range(_DIM):
                val = plsc.load_gather(rows_v.at[buf], [jv, off + d])
                tr_v[obuf, d // 8, d % 8, pl.ds(g * _L, _L)] = val

        pltpu.async_copy(tr_v.at[obuf], out_hbm.at[t, :, wid], osem)
        return carry

    lax.fori_loop(0, _SEQ, step, 0)

    pltpu.make_async_copy(
        tr_v.at[0], out_hbm.at[_SEQ - 2, :, wid], osem).wait()
    pltpu.make_async_copy(
        tr_v.at[1], out_hbm.at[_SEQ - 1, :, wid], osem).wait()


def kernel(X, W):
    xi = X.astype(jnp.int32)
    idxs = (xi >> 2).T
    idxm = (xi & 3).T
    w4 = W.reshape(_BATCH // _BATCH * 250000, 128)
    out5 = _gather_kernel(idxs, idxm, w4)
    return out5.transpose(2, 4, 0, 1, 3).reshape(_BATCH, _SEQ, _DIM)
